# baseline (speedup 1.0000x reference)
import functools

import jax
import jax.numpy as jnp
from jax import lax
from jax.experimental import pallas as pl
from jax.experimental.pallas import tpu as pltpu
from jax.experimental.pallas import tpu_sc as plsc

N = 10000
E = 320000
D = 128
H = 128
TWOH = 256
G = 128
OUT = 10

_NC = 2
_NS = 16
_NW = _NC * _NS
_C = 80
_NBUF = 2
_ITERS = 128
_EPW = _ITERS * _C
_EPAD = _NW * _EPW
_NPAD = N + 16
_RPS = 624
_TAIL = N - _NS * _RPS


def _make_seg_sum():
    mesh = plsc.VectorSubcoreMesh(core_axis_name="c", subcore_axis_name="s")

    @functools.partial(
        pl.kernel,
        out_type=jax.ShapeDtypeStruct((_NC, N, D), jnp.float32),
        mesh=mesh,
        scratch_types=[
            pltpu.VMEM((_C,), jnp.int32),
            pltpu.VMEM((_C,), jnp.int32),
            pltpu.VMEM((_C,), jnp.int32),
            pltpu.VMEM((_C,), jnp.int32),
            pltpu.VMEM((_C, D), jnp.float32),
            pltpu.VMEM((_C, D), jnp.float32),
            pltpu.VMEM_SHARED((_NPAD, D), jnp.float32),
            pltpu.SemaphoreType.DMA,
            pltpu.SemaphoreType.DMA,
        ],
    )
    def seg_sum(h_hbm, src_hbm, dst_hbm, zeros_hbm, out_hbm,
                sidx0, didx0, sidx1, didx1, rows0, rows1, agg_sh, sem0, sem1):
        sidx = [sidx0, sidx1]
        didx = [didx0, didx1]
        rows = [rows0, rows1]
        sems = [sem0, sem1]
        cid = lax.axis_index("c")
        sid = lax.axis_index("s")
        wid = sid * _NC + cid
        pltpu.sync_copy(zeros_hbm.at[pl.ds(sid * _RPS, _RPS)],
                        agg_sh.at[pl.ds(sid * _RPS, _RPS)])
        @pl.when(sid == 0)
        def _():
            pltpu.sync_copy(zeros_hbm.at[pl.ds(_NS * _RPS, _TAIL)],
                            agg_sh.at[pl.ds(_NS * _RPS, _TAIL)])
        plsc.subcore_barrier()

        base = wid * _EPW

        def stage(slot, j):
            off = base + j * _C
            pltpu.sync_copy(src_hbm.at[pl.ds(off, _C)], sidx[slot])
            pltpu.sync_copy(dst_hbm.at[pl.ds(off, _C)], didx[slot])
            pltpu.async_copy(h_hbm.at[sidx[slot]], rows[slot], sems[slot])

        def drain_scatter(b):
            pltpu.make_async_copy(
                h_hbm.at[pl.ds(0, _C)], rows[b], sems[b]).wait()
            pltpu.sync_copy(rows[b], agg_sh.at[didx[b]], add=True)

        stage(0, 0)

        def body(i, carry):
            j0 = i * _NBUF
            for b in range(_NBUF):
                stage((b + 1) % _NBUF, j0 + b + 1)
                drain_scatter(b)
            return carry

        lax.fori_loop(0, _ITERS // _NBUF - 1, body, 0)
        stage(1, _ITERS - 1)
        drain_scatter(0)
        drain_scatter(1)
        plsc.subcore_barrier()
        pltpu.sync_copy(agg_sh.at[pl.ds(sid * _RPS, _RPS)],
                        out_hbm.at[cid, pl.ds(sid * _RPS, _RPS)])
        @pl.when(sid == 0)
        def _():
            pltpu.sync_copy(agg_sh.at[pl.ds(_NS * _RPS, _TAIL)],
                            out_hbm.at[cid, pl.ds(_NS * _RPS, _TAIL)])

    return seg_sum


_seg_sum_cache = []


def _seg_sum(h, src, dst, zeros):
    if not _seg_sum_cache:
        _seg_sum_cache.append(_make_seg_sum())
    return _seg_sum_cache[0](h, src, dst, zeros)


def _bn_relu(t, g, beta):
    mu = jnp.mean(t, axis=0, keepdims=True)
    var = jnp.mean((t - mu) * (t - mu), axis=0, keepdims=True)
    tn = (t - mu) / jnp.sqrt(var + 1e-5) * g + beta
    return jnp.maximum(tn, 0.0)


def _gin_dense_body(h_ref, a0_ref, a1_ref, sc_ref, w1_ref, b1_ref, g1_ref,
                    be1_ref, w2_ref, b2_ref, out_ref):
    a = sc_ref[0, 0] * h_ref[...] + a0_ref[...] + a1_ref[...]
    t = jnp.dot(a, w1_ref[...], preferred_element_type=jnp.float32)
    t = t + b1_ref[...]
    tn = _bn_relu(t, g1_ref[...], be1_ref[...])
    h2 = jnp.dot(tn, w2_ref[...], preferred_element_type=jnp.float32)
    h2 = h2 + b2_ref[...]
    out_ref[...] = jnp.maximum(h2, 0.0)


def _gin_dense(h, a0, a1, scale, p):
    return pl.pallas_call(
        _gin_dense_body,
        out_shape=jax.ShapeDtypeStruct((N, H), jnp.float32),
    )(h, a0, a1, scale,
      p['W1'], p['b1'].reshape(1, TWOH), p['g1'].reshape(1, TWOH),
      p['beta1'].reshape(1, TWOH), p['W2'], p['b2'].reshape(1, H))


def _head_body(h_ref, a0_ref, a1_ref, sc_ref, w1_ref, b1_ref, g1_ref,
               be1_ref, w2_ref, b2_ref, batch_ref, wo1_ref, bo1_ref, go1_ref,
               beo1_ref, wo2_ref, bo2_ref, go2_ref, beo2_ref, wo3_ref,
               bo3_ref, out_ref):
    a = sc_ref[0, 0] * h_ref[...] + a0_ref[...] + a1_ref[...]
    t = jnp.dot(a, w1_ref[...], preferred_element_type=jnp.float32)
    t = t + b1_ref[...]
    tn = _bn_relu(t, g1_ref[...], be1_ref[...])
    h2 = jnp.dot(tn, w2_ref[...], preferred_element_type=jnp.float32)
    h3 = jnp.maximum(h2 + b2_ref[...], 0.0)
    gid = lax.broadcasted_iota(jnp.int32, (G, N), 0)
    onehot_t = (gid == batch_ref[...]).astype(jnp.float32)
    pooled = jnp.dot(onehot_t, h3, preferred_element_type=jnp.float32,
                     precision=lax.Precision.HIGHEST)
    counts = jnp.dot(onehot_t, jnp.ones((N, 1), jnp.float32),
                     preferred_element_type=jnp.float32,
                     precision=lax.Precision.HIGHEST)
    gr = pooled / jnp.maximum(counts, 1.0)
    o = jnp.dot(gr, wo1_ref[...], preferred_element_type=jnp.float32)
    o = _bn_relu(o + bo1_ref[...], go1_ref[...], beo1_ref[...])
    o = jnp.dot(o, wo2_ref[...], preferred_element_type=jnp.float32)
    o = _bn_relu(o + bo2_ref[...], go2_ref[...], beo2_ref[...])
    logits = jnp.dot(o, wo3_ref[...], preferred_element_type=jnp.float32)
    logits = logits + bo3_ref[...]
    m = jnp.max(logits, axis=1, keepdims=True)
    e = jnp.exp(logits - m)
    out_ref[...] = e / jnp.sum(e, axis=1, keepdims=True)


def _head(h, a0, a1, scale, p, batch_row, po):
    return pl.pallas_call(
        _head_body,
        out_shape=jax.ShapeDtypeStruct((G, OUT), jnp.float32),
    )(h, a0, a1, scale,
      p['W1'], p['b1'].reshape(1, TWOH), p['g1'].reshape(1, TWOH),
      p['beta1'].reshape(1, TWOH), p['W2'], p['b2'].reshape(1, H),
      batch_row,
      po['W1'], po['b1'].reshape(1, TWOH), po['g1'].reshape(1, TWOH),
      po['beta1'].reshape(1, TWOH),
      po['W2'], po['b2'].reshape(1, H), po['g2'].reshape(1, H),
      po['beta2'].reshape(1, H),
      po['W3'], po['b3'].reshape(1, OUT))


def kernel(x, edge_index, batch, params):
    pad = _EPAD - E
    src = jnp.concatenate([edge_index[0], jnp.zeros((pad,), jnp.int32)])
    dst = jnp.concatenate([edge_index[1], jnp.full((pad,), N, jnp.int32)])
    zeros = jnp.zeros((N, D), jnp.float32)
    batch_row = batch.reshape(1, N)

    h = x
    for l, p in enumerate(params['gin']):
        agg = _seg_sum(h, src, dst, zeros)
        scale = (1.0 + p['eps']).reshape(1, 1)
        if l < 2:
            h = _gin_dense(h, agg[0], agg[1], scale, p)
        else:
            y = _head(h, agg[0], agg[1], scale, p, batch_row, params['out'])
    return (y, y)


# speedup vs baseline: 1.0012x; 1.0012x over previous
import functools

import jax
import jax.numpy as jnp
from jax import lax
from jax.experimental import pallas as pl
from jax.experimental.pallas import tpu as pltpu
from jax.experimental.pallas import tpu_sc as plsc

N = 10000
E = 320000
D = 128
H = 128
TWOH = 256
G = 128
OUT = 10

_NC = 2
_NS = 16
_NW = _NC * _NS
_C = 80
_NBUF = 2
_ITERS = 128
_EPW = _ITERS * _C
_EPAD = _NW * _EPW
_NPAD = N + 16
_RPS = 624
_TAIL = N - _NS * _RPS


def _make_seg_sum():
    mesh = plsc.VectorSubcoreMesh(core_axis_name="c", subcore_axis_name="s")

    @functools.partial(
        pl.kernel,
        out_type=jax.ShapeDtypeStruct((_NC, N, D), jnp.float32),
        mesh=mesh,
        scratch_types=[
            pltpu.VMEM((_C,), jnp.int32),
            pltpu.VMEM((_C,), jnp.int32),
            pltpu.VMEM((_C,), jnp.int32),
            pltpu.VMEM((_C,), jnp.int32),
            pltpu.VMEM((_C, D), jnp.float32),
            pltpu.VMEM((_C, D), jnp.float32),
            pltpu.VMEM_SHARED((_NPAD, D), jnp.float32),
            pltpu.SemaphoreType.DMA,
            pltpu.SemaphoreType.DMA,
        ],
    )
    def seg_sum(h_hbm, src_hbm, dst_hbm, zeros_hbm, out_hbm,
                sidx0, didx0, sidx1, didx1, rows0, rows1, agg_sh, sem0, sem1):
        sidx = [sidx0, sidx1]
        didx = [didx0, didx1]
        rows = [rows0, rows1]
        sems = [sem0, sem1]
        cid = lax.axis_index("c")
        sid = lax.axis_index("s")
        wid = sid * _NC + cid
        pltpu.sync_copy(zeros_hbm.at[pl.ds(sid * _RPS, _RPS)],
                        agg_sh.at[pl.ds(sid * _RPS, _RPS)])
        @pl.when(sid == 0)
        def _():
            pltpu.sync_copy(zeros_hbm.at[pl.ds(_NS * _RPS, _TAIL)],
                            agg_sh.at[pl.ds(_NS * _RPS, _TAIL)])
        plsc.subcore_barrier()

        base = wid * _EPW

        def stage(slot, j):
            off = base + j * _C
            pltpu.sync_copy(src_hbm.at[pl.ds(off, _C)], sidx[slot])
            pltpu.sync_copy(dst_hbm.at[pl.ds(off, _C)], didx[slot])
            pltpu.async_copy(h_hbm.at[sidx[slot]], rows[slot], sems[slot])

        def drain_scatter(b):
            pltpu.make_async_copy(
                h_hbm.at[pl.ds(0, _C)], rows[b], sems[b]).wait()
            pltpu.sync_copy(rows[b], agg_sh.at[didx[b]], add=True)

        stage(0, 0)

        def body(i, carry):
            j0 = i * _NBUF
            for b in range(_NBUF):
                stage((b + 1) % _NBUF, j0 + b + 1)
                drain_scatter(b)
            return carry

        lax.fori_loop(0, _ITERS // _NBUF - 1, body, 0)
        stage(1, _ITERS - 1)
        drain_scatter(0)
        drain_scatter(1)
        plsc.subcore_barrier()
        pltpu.sync_copy(agg_sh.at[pl.ds(sid * _RPS, _RPS)],
                        out_hbm.at[cid, pl.ds(sid * _RPS, _RPS)])
        @pl.when(sid == 0)
        def _():
            pltpu.sync_copy(agg_sh.at[pl.ds(_NS * _RPS, _TAIL)],
                            out_hbm.at[cid, pl.ds(_NS * _RPS, _TAIL)])

    return seg_sum


_seg_sum_cache = []


def _seg_sum(h, src, dst, zeros):
    if not _seg_sum_cache:
        _seg_sum_cache.append(_make_seg_sum())
    return _seg_sum_cache[0](h, src, dst, zeros)


def _bn_relu(t, g, beta):
    mu = jnp.mean(t, axis=0, keepdims=True)
    var = jnp.mean((t - mu) * (t - mu), axis=0, keepdims=True)
    tn = (t - mu) / jnp.sqrt(var + 1e-5) * g + beta
    return jnp.maximum(tn, 0.0)


def _gin_dense_body(h_ref, a0_ref, a1_ref, sc_ref, w1_ref, b1_ref, g1_ref,
                    be1_ref, w2_ref, b2_ref, out_ref):
    a = sc_ref[0, 0] * h_ref[...] + a0_ref[...] + a1_ref[...]
    t = jnp.dot(a, w1_ref[...], preferred_element_type=jnp.float32)
    t = t + b1_ref[...]
    tn = _bn_relu(t, g1_ref[...], be1_ref[...])
    h2 = jnp.dot(tn, w2_ref[...], preferred_element_type=jnp.float32)
    h2 = h2 + b2_ref[...]
    out_ref[...] = jnp.maximum(h2, 0.0)


def _gin_dense(h, a0, a1, scale, p):
    return pl.pallas_call(
        _gin_dense_body,
        out_shape=jax.ShapeDtypeStruct((N, H), jnp.float32),
    )(h, a0, a1, scale,
      p['W1'], p['b1'].reshape(1, TWOH), p['g1'].reshape(1, TWOH),
      p['beta1'].reshape(1, TWOH), p['W2'], p['b2'].reshape(1, H))


def _head_body(h_ref, a0_ref, a1_ref, sc_ref, w1_ref, b1_ref, g1_ref,
               be1_ref, w2_ref, b2_ref, batch_ref, wo1_ref, bo1_ref, go1_ref,
               beo1_ref, wo2_ref, bo2_ref, go2_ref, beo2_ref, wo3_ref,
               bo3_ref, out_ref):
    a = sc_ref[0, 0] * h_ref[...] + a0_ref[...] + a1_ref[...]
    t = jnp.dot(a, w1_ref[...], preferred_element_type=jnp.float32)
    t = t + b1_ref[...]
    tn = _bn_relu(t, g1_ref[...], be1_ref[...])
    h2 = jnp.dot(tn, w2_ref[...], preferred_element_type=jnp.float32)
    h3 = jnp.maximum(h2 + b2_ref[...], 0.0)
    gid = lax.broadcasted_iota(jnp.int32, (G, N), 0)
    onehot_t = (gid == batch_ref[...]).astype(jnp.float32)
    pooled = jnp.dot(onehot_t, h3, preferred_element_type=jnp.float32,
                     precision=lax.Precision.HIGHEST)
    counts = jnp.dot(onehot_t, jnp.ones((N, 1), jnp.float32),
                     preferred_element_type=jnp.float32,
                     precision=lax.Precision.HIGHEST)
    gr = pooled / jnp.maximum(counts, 1.0)
    o = jnp.dot(gr, wo1_ref[...], preferred_element_type=jnp.float32)
    o = _bn_relu(o + bo1_ref[...], go1_ref[...], beo1_ref[...])
    o = jnp.dot(o, wo2_ref[...], preferred_element_type=jnp.float32)
    o = _bn_relu(o + bo2_ref[...], go2_ref[...], beo2_ref[...])
    logits = jnp.dot(o, wo3_ref[...], preferred_element_type=jnp.float32)
    logits = logits + bo3_ref[...]
    m = jnp.max(logits, axis=1, keepdims=True)
    e = jnp.exp(logits - m)
    out_ref[...] = e / jnp.sum(e, axis=1, keepdims=True)


def _head(h, a0, a1, scale, p, batch_row, po):
    return pl.pallas_call(
        _head_body,
        out_shape=jax.ShapeDtypeStruct((G, OUT), jnp.float32),
    )(h, a0, a1, scale,
      p['W1'], p['b1'].reshape(1, TWOH), p['g1'].reshape(1, TWOH),
      p['beta1'].reshape(1, TWOH), p['W2'], p['b2'].reshape(1, H),
      batch_row,
      po['W1'], po['b1'].reshape(1, TWOH), po['g1'].reshape(1, TWOH),
      po['beta1'].reshape(1, TWOH),
      po['W2'], po['b2'].reshape(1, H), po['g2'].reshape(1, H),
      po['beta2'].reshape(1, H),
      po['W3'], po['b3'].reshape(1, OUT))


def kernel(x, edge_index, batch, params):
    pad = _EPAD - E
    pad_dst = N + (jnp.arange(pad, dtype=jnp.int32) % (_NPAD - N))
    src = jnp.concatenate([edge_index[0], jnp.zeros((pad,), jnp.int32)])
    dst = jnp.concatenate([edge_index[1], pad_dst])
    zeros = jnp.zeros((N, D), jnp.float32)
    batch_row = batch.reshape(1, N)

    h = x
    for l, p in enumerate(params['gin']):
        agg = _seg_sum(h, src, dst, zeros)
        scale = (1.0 + p['eps']).reshape(1, 1)
        if l < 2:
            h = _gin_dense(h, agg[0], agg[1], scale, p)
        else:
            y = _head(h, agg[0], agg[1], scale, p, batch_row, params['out'])
    return (y, y)


# speedup vs baseline: 2.3192x; 2.3164x over previous
import functools

import jax
import jax.numpy as jnp
from jax import lax
from jax.experimental import pallas as pl
from jax.experimental.pallas import tpu as pltpu
from jax.experimental.pallas import tpu_sc as plsc

N = 10000
E = 320000
D = 128
H = 128
TWOH = 256
G = 128
OUT = 10

_NC = 2
_NS = 16
_NW = _NC * _NS
_C = 80
_NBUF = 2
_ITERS = 128
_EPW = _ITERS * _C
_EPAD = _NW * _EPW
_NPAD = N + 16
_RPS = 624
_TAIL = N - _NS * _RPS


def _make_seg_sum():
    mesh = plsc.VectorSubcoreMesh(core_axis_name="c", subcore_axis_name="s")

    @functools.partial(
        pl.kernel,
        out_type=jax.ShapeDtypeStruct((_NC, N, D), jnp.float32),
        mesh=mesh,
        scratch_types=[
            pltpu.VMEM((_C,), jnp.int32),
            pltpu.VMEM((_C,), jnp.int32),
            pltpu.VMEM((_C,), jnp.int32),
            pltpu.VMEM((_C,), jnp.int32),
            pltpu.VMEM((_C, D), jnp.float32),
            pltpu.VMEM((_C, D), jnp.float32),
            pltpu.VMEM_SHARED((_NPAD, D), jnp.float32),
            pltpu.SemaphoreType.DMA,
            pltpu.SemaphoreType.DMA,
        ],
    )
    def seg_sum(h_hbm, src_hbm, dst_hbm, zeros_hbm, out_hbm,
                sidx0, didx0, sidx1, didx1, rows0, rows1, agg_sh, sem0, sem1):
        sidx = [sidx0, sidx1]
        didx = [didx0, didx1]
        rows = [rows0, rows1]
        sems = [sem0, sem1]
        cid = lax.axis_index("c")
        sid = lax.axis_index("s")
        wid = sid * _NC + cid
        pltpu.sync_copy(zeros_hbm.at[pl.ds(sid * _RPS, _RPS)],
                        agg_sh.at[pl.ds(sid * _RPS, _RPS)])
        @pl.when(sid == 0)
        def _():
            pltpu.sync_copy(zeros_hbm.at[pl.ds(_NS * _RPS, _TAIL)],
                            agg_sh.at[pl.ds(_NS * _RPS, _TAIL)])
        plsc.subcore_barrier()

        base = wid * _EPW

        def stage(slot, j):
            off = base + j * _C
            pltpu.sync_copy(src_hbm.at[pl.ds(off, _C)], sidx[slot])
            pltpu.sync_copy(dst_hbm.at[pl.ds(off, _C)], didx[slot])
            pltpu.async_copy(h_hbm.at[sidx[slot]], rows[slot], sems[slot])

        def drain_scatter(b):
            pltpu.make_async_copy(
                h_hbm.at[pl.ds(0, _C)], rows[b], sems[b]).wait()
            pltpu.sync_copy(rows[b], agg_sh.at[didx[b]], add=True)

        stage(0, 0)

        def body(i, carry):
            j0 = i * _NBUF
            for b in range(_NBUF):
                stage((b + 1) % _NBUF, j0 + b + 1)
                drain_scatter(b)
            return carry

        lax.fori_loop(0, _ITERS // _NBUF - 1, body, 0)
        stage(1, _ITERS - 1)
        drain_scatter(0)
        drain_scatter(1)
        plsc.subcore_barrier()
        pltpu.sync_copy(agg_sh.at[pl.ds(sid * _RPS, _RPS)],
                        out_hbm.at[cid, pl.ds(sid * _RPS, _RPS)])
        @pl.when(sid == 0)
        def _():
            pltpu.sync_copy(agg_sh.at[pl.ds(_NS * _RPS, _TAIL)],
                            out_hbm.at[cid, pl.ds(_NS * _RPS, _TAIL)])

    return seg_sum


_seg_sum_cache = []


def _seg_sum(h, src, dst, zeros):
    if not _seg_sum_cache:
        _seg_sum_cache.append(_make_seg_sum())
    return _seg_sum_cache[0](h, src, dst, zeros)


def _bn_relu(t, g, beta):
    mu = jnp.mean(t, axis=0, keepdims=True)
    var = jnp.mean((t - mu) * (t - mu), axis=0, keepdims=True)
    tn = (t - mu) / jnp.sqrt(var + 1e-5) * g + beta
    return jnp.maximum(tn, 0.0)


def _gin_dense_body(h_ref, a0_ref, a1_ref, sc_ref, w1_ref, b1_ref, g1_ref,
                    be1_ref, w2_ref, b2_ref, out_ref):
    a = sc_ref[0, 0] * h_ref[...] + a0_ref[...] + a1_ref[...]
    t = jnp.dot(a, w1_ref[...], preferred_element_type=jnp.float32)
    t = t + b1_ref[...]
    tn = _bn_relu(t, g1_ref[...], be1_ref[...])
    h2 = jnp.dot(tn, w2_ref[...], preferred_element_type=jnp.float32)
    h2 = h2 + b2_ref[...]
    out_ref[...] = jnp.maximum(h2, 0.0)


def _gin_dense(h, a0, a1, scale, p):
    return pl.pallas_call(
        _gin_dense_body,
        out_shape=jax.ShapeDtypeStruct((N, H), jnp.float32),
    )(h, a0, a1, scale,
      p['W1'], p['b1'].reshape(1, TWOH), p['g1'].reshape(1, TWOH),
      p['beta1'].reshape(1, TWOH), p['W2'], p['b2'].reshape(1, H))


def _head_body(h_ref, a0_ref, a1_ref, sc_ref, w1_ref, b1_ref, g1_ref,
               be1_ref, w2_ref, b2_ref, batch_ref, wo1_ref, bo1_ref, go1_ref,
               beo1_ref, wo2_ref, bo2_ref, go2_ref, beo2_ref, wo3_ref,
               bo3_ref, out_ref):
    a = sc_ref[0, 0] * h_ref[...] + a0_ref[...] + a1_ref[...]
    t = jnp.dot(a, w1_ref[...], preferred_element_type=jnp.float32)
    t = t + b1_ref[...]
    tn = _bn_relu(t, g1_ref[...], be1_ref[...])
    h2 = jnp.dot(tn, w2_ref[...], preferred_element_type=jnp.float32)
    h3 = jnp.maximum(h2 + b2_ref[...], 0.0)
    gid = lax.broadcasted_iota(jnp.int32, (G, N), 0)
    onehot_t = (gid == batch_ref[...]).astype(jnp.float32)
    pooled = jnp.dot(onehot_t, h3, preferred_element_type=jnp.float32,
                     precision=lax.Precision.HIGHEST)
    counts = jnp.dot(onehot_t, jnp.ones((N, 1), jnp.float32),
                     preferred_element_type=jnp.float32,
                     precision=lax.Precision.HIGHEST)
    gr = pooled / jnp.maximum(counts, 1.0)
    o = jnp.dot(gr, wo1_ref[...], preferred_element_type=jnp.float32)
    o = _bn_relu(o + bo1_ref[...], go1_ref[...], beo1_ref[...])
    o = jnp.dot(o, wo2_ref[...], preferred_element_type=jnp.float32)
    o = _bn_relu(o + bo2_ref[...], go2_ref[...], beo2_ref[...])
    logits = jnp.dot(o, wo3_ref[...], preferred_element_type=jnp.float32)
    logits = logits + bo3_ref[...]
    m = jnp.max(logits, axis=1, keepdims=True)
    e = jnp.exp(logits - m)
    out_ref[...] = e / jnp.sum(e, axis=1, keepdims=True)


def _head(h, a0, a1, scale, p, batch_row, po):
    return pl.pallas_call(
        _head_body,
        out_shape=jax.ShapeDtypeStruct((G, OUT), jnp.float32),
    )(h, a0, a1, scale,
      p['W1'], p['b1'].reshape(1, TWOH), p['g1'].reshape(1, TWOH),
      p['beta1'].reshape(1, TWOH), p['W2'], p['b2'].reshape(1, H),
      batch_row,
      po['W1'], po['b1'].reshape(1, TWOH), po['g1'].reshape(1, TWOH),
      po['beta1'].reshape(1, TWOH),
      po['W2'], po['b2'].reshape(1, H), po['g2'].reshape(1, H),
      po['beta2'].reshape(1, H),
      po['W3'], po['b3'].reshape(1, OUT))


def kernel(x, edge_index, batch, params):
    pad = _EPAD - E
    pad_dst = N + (jnp.arange(pad, dtype=jnp.int32) % (_NPAD - N))
    pad_src = jnp.arange(pad, dtype=jnp.int32) % N
    src = jnp.concatenate([edge_index[0], pad_src])
    dst = jnp.concatenate([edge_index[1], pad_dst])
    zeros = jnp.zeros((N, D), jnp.float32)
    batch_row = batch.reshape(1, N)

    h = x
    for l, p in enumerate(params['gin']):
        agg = _seg_sum(h, src, dst, zeros)
        scale = (1.0 + p['eps']).reshape(1, 1)
        if l < 2:
            h = _gin_dense(h, agg[0], agg[1], scale, p)
        else:
            y = _head(h, agg[0], agg[1], scale, p, batch_row, params['out'])
    return (y, y)
